# baseline (speedup 1.0000x reference)
import functools

import jax
import jax.numpy as jnp
from jax import lax
from jax.experimental import pallas as pl
from jax.experimental.pallas import tpu as pltpu
from jax.experimental.pallas import tpu_sc as plsc

_NW = 32
_PASS = 96
_L = 16


def _sc_dots_body(gae, gas, gbe, gbs, gm, ra_h, rb_h, ca_h, cb_h, cm_h,
                  ent_in, ent_out, small, out,
                  vae, vas, vbe, vbs, vm, vra, vrb, vca, vcb, vcm,
                  bufa, bufb, bufm, dots,
                  s0, s1, s2, s3, s4, *, ch, emb):
    wid = lax.axis_index("s") * 2 + lax.axis_index("c")
    base = wid * ch
    sl_all = pl.ds(base, ch)
    pltpu.sync_copy(gae.at[sl_all], vae)
    pltpu.sync_copy(gas.at[sl_all], vas)
    pltpu.sync_copy(gbe.at[sl_all], vbe)
    pltpu.sync_copy(gbs.at[sl_all], vbs)
    pltpu.sync_copy(gm.at[sl_all], vm)
    pltpu.sync_copy(ra_h.at[sl_all], vra)
    pltpu.sync_copy(rb_h.at[sl_all], vrb)
    pltpu.sync_copy(ca_h.at[sl_all], vca)
    pltpu.sync_copy(cb_h.at[sl_all], vcb)
    pltpu.sync_copy(cm_h.at[sl_all], vcm)
    zero = jnp.zeros((_L,), jnp.float32)
    for p in range(ch // _PASS):
        sl = pl.ds(p * _PASS, _PASS)
        cs = [pltpu.async_copy(ent_in.at[vae.at[sl]], bufa.at[pl.ds(0, _PASS)], s0),
              pltpu.async_copy(small.at[vas.at[sl]], bufa.at[pl.ds(_PASS, _PASS)], s1),
              pltpu.async_copy(ent_out.at[vbe.at[sl]], bufb.at[pl.ds(0, _PASS)], s2),
              pltpu.async_copy(small.at[vbs.at[sl]], bufb.at[pl.ds(_PASS, _PASS)], s3),
              pltpu.async_copy(small.at[vm.at[sl]], bufm, s4)]
        for c in cs:
            c.wait()
        for g in range(_PASS // _L):
            s0_ = p * _PASS + g * _L
            lsl = pl.ds(s0_, _L)
            ra = vra[lsl]
            rb = vrb[lsl]
            rm = lax.iota(jnp.int32, _L) + (g * _L)
            ca = vca[lsl]
            cb = vcb[lsl]
            cm = vcm[lsl]
            pab, pam, pbm = zero, zero, zero
            for d in range(emb):
                va = plsc.load_gather(bufa, [ra, ca + d])
                vb = plsc.load_gather(bufb, [rb, cb + d])
                vmm = plsc.load_gather(bufm, [rm, cm + d])
                pab = pab + va * vb
                pam = pam + va * vmm
                pbm = pbm + vb * vmm
            dots[lsl] = pab - pam * pbm
    pltpu.sync_copy(dots, out.at[sl_all])


def _make_sc_dots(n, lanes, emb):
    ch = n // _NW
    mesh = plsc.VectorSubcoreMesh(core_axis_name="c", subcore_axis_name="s")
    return pl.kernel(
        functools.partial(_sc_dots_body, ch=ch, emb=emb),
        mesh=mesh,
        out_type=jax.ShapeDtypeStruct((n,), jnp.float32),
        scratch_types=[pltpu.VMEM((ch,), jnp.int32)] * 10
        + [pltpu.VMEM((2 * _PASS, lanes), jnp.float32)] * 2
        + [pltpu.VMEM((_PASS, lanes), jnp.float32),
           pltpu.VMEM((ch,), jnp.float32)]
        + [pltpu.SemaphoreType.DMA] * 5,
        compiler_params=pltpu.CompilerParams(needs_layout_passes=False),
    )


def _log_sigmoid(x):
    return jnp.minimum(x, 0.0) - jnp.log(1.0 + jnp.exp(-jnp.abs(x)))


def _tc_loss_body(dp, dn, out):
    acc = _log_sigmoid(dp[...]) + jnp.sum(_log_sigmoid(-dn[...]),
                                          axis=1, keepdims=True)
    out[...] = -acc


def _prep(labels_in, labels_tgt, ne, rel, nq):
    ie = labels_in < ne
    te = labels_tgt < ne
    io = jnp.where(ie, labels_in, labels_in - ne).astype(jnp.int32)
    to = jnp.where(te, labels_tgt, labels_tgt - ne).astype(jnp.int32)
    case = jnp.where(ie & te, 0, jnp.where(ie & (~te), 1,
                     jnp.where((~ie) & te, 2, 3)))
    perm = jnp.argsort(case)
    io, to, ie, te, case = io[perm], to[perm], ie[perm], te[perm], case[perm]

    zg = (4 * rel) // nq
    gae = jnp.where(ie, io // nq, 0)
    gas = jnp.where(ie, 0, io // nq)
    gbe = jnp.where(te, to // nq, 0)
    gbs = jnp.where(te, 0, (rel + to) // nq)
    m_act = (case == 1) | (case == 2)
    row_m = jnp.where(case == 1, 2 * rel + to, 3 * rel + io)
    gm = jnp.where(m_act, row_m // nq, zg)
    sel_a = jnp.where(ie, 0, _PASS).astype(jnp.int32)
    sel_b = jnp.where(te, 0, _PASS).astype(jnp.int32)
    ca = (io % nq) * (128 // nq)
    cb = (to % nq) * (128 // nq)
    cm = jnp.where(m_act, (row_m % nq) * (128 // nq), 0)
    return gae, gas, gbe, gbs, gm, sel_a, sel_b, ca, cb, cm


def kernel(input_labels, pos_labels, neg_labels, ent_dic, reverse_dictionary,
           in_embed_ent, out_embed_ent, in_embed_rel, out_embed_rel,
           in_embed_map, out_embed_map):
    b = input_labels.shape[0]
    k = neg_labels.shape[0] // b
    emb = in_embed_ent.shape[1]
    ne = ent_dic.shape[0]
    rel = in_embed_rel.shape[0]
    n = b * (k + 1)
    nq = 128 // emb
    lanes = 128

    p = _prep(input_labels.reshape(-1), pos_labels.reshape(-1), ne, rel, nq)
    q = _prep(jnp.repeat(input_labels.reshape(-1), k),
              neg_labels.reshape(-1), ne, rel, nq)
    cat = [jnp.concatenate([pi, qi]) for pi, qi in zip(p, q)]
    gae, gas, gbe, gbs, gm, sel_a, sel_b, ca, cb, cm = cat
    pos_in_pass = (jnp.arange(n, dtype=jnp.int32) % _PASS)
    ra = sel_a + pos_in_pass
    rb = sel_b + pos_in_pass

    ent_in = in_embed_ent.reshape(-1, lanes)
    ent_out = out_embed_ent.reshape(-1, lanes)
    small = jnp.concatenate(
        [in_embed_rel, out_embed_rel, in_embed_map, out_embed_map,
         jnp.zeros((nq, emb), jnp.float32)], axis=0).reshape(-1, lanes)

    dots = _make_sc_dots(n, lanes, emb)(
        gae, gas, gbe, gbs, gm, ra, rb, ca, cb, cm,
        ent_in, ent_out, small)

    loss2d = pl.pallas_call(
        _tc_loss_body,
        out_shape=jax.ShapeDtypeStruct((b, 1), jnp.float32),
    )(dots[:b].reshape(b, 1), dots[b:].reshape(b, k))
    return loss2d.reshape(b)
